# baseline (speedup 1.0000x reference)
import jax
import jax.numpy as jnp
from jax.experimental import pallas as pl
from jax.experimental.pallas import tpu as pltpu

_E = 8
_C = 128
_NY = 64
_NX = 128
_P = 2
_N = _NY * _NX
_T = 512


def _fused(x_ref, eta_ref, W_ref, b_ref, Wt_ref, Tmat_ref, out_ref, eta_out_ref):
    x_t = x_ref[...]
    eta_t = eta_ref[0, :]

    logits = jax.lax.dot_general(
        x_t, Wt_ref[...], (((0,), (0,)), ((), ())),
        preferred_element_type=jnp.float32)
    tadd = jnp.zeros((_T, _E), jnp.float32)
    for k in range(_E):
        tadd = jnp.where(eta_t[:, None] == k, Tmat_ref[k][None, :], tadd)
    logits = logits + tadd
    new_eta = jnp.argmax(logits, axis=1).astype(jnp.int32)
    eta_out_ref[0, :] = new_eta

    mask = (new_eta[None, :] == jax.lax.broadcasted_iota(
        jnp.int32, (_E, _T), 0)).astype(jnp.float32)
    mask_bf = mask.astype(jnp.bfloat16)
    xm = jnp.concatenate(
        [x_t * mask_bf[e:e + 1, :] for e in range(_E)], axis=0)
    badd = jax.lax.dot_general(
        b_ref[...], mask, (((2,), (0,)), ((), ())),
        preferred_element_type=jnp.float32)

    for p in range(_P):
        y = jax.lax.dot_general(
            W_ref[p], xm, (((1,), (0,)), ((), ())),
            preferred_element_type=jnp.float32)
        out_ref[p] = y + badd[p]


def kernel(x, eta, W, b, Wt, Tmat):
    x2 = x.reshape(_C, _N).astype(jnp.bfloat16)
    eta2 = eta.reshape(1, _N).astype(jnp.int32)
    W2 = jnp.transpose(W, (0, 2, 1, 3)).reshape(_P, _C, _E * _C)
    W2 = W2.astype(jnp.bfloat16)
    b2 = jnp.transpose(b, (0, 2, 1))
    Wt2 = Wt.astype(jnp.bfloat16)
    grid = (_N // _T,)
    out, new_eta = pl.pallas_call(
        _fused,
        grid=grid,
        in_specs=[
            pl.BlockSpec((_C, _T), lambda i: (0, i)),
            pl.BlockSpec((1, _T), lambda i: (0, i)),
            pl.BlockSpec((_P, _C, _E * _C), lambda i: (0, 0, 0)),
            pl.BlockSpec((_P, _C, _E), lambda i: (0, 0, 0)),
            pl.BlockSpec((_C, _E), lambda i: (0, 0)),
            pl.BlockSpec((_E, _E), lambda i: (0, 0)),
        ],
        out_specs=[
            pl.BlockSpec((_P, _C, _T), lambda i: (0, 0, i)),
            pl.BlockSpec((1, _T), lambda i: (0, i)),
        ],
        out_shape=[
            jax.ShapeDtypeStruct((_P, _C, _N), jnp.float32),
            jax.ShapeDtypeStruct((1, _N), jnp.int32),
        ],
        compiler_params=pltpu.CompilerParams(
            dimension_semantics=("arbitrary",)),
    )(x2, eta2, W2, b2, Wt2, Tmat)
    return out.reshape(_P, _C, _NY, _NX), new_eta.reshape(_NY, _NX)
